# baseline (speedup 1.0000x reference)
import jax
import jax.numpy as jnp
from jax import lax
from jax.experimental import pallas as pl
from jax.experimental.pallas import tpu as pltpu
from jax.experimental.pallas import tpu_sc as plsc

NS = 16
D = 128
CH = 128


def _k1_body(nodes_r, cols_r, emb, feat_out, deg_out,
             idx_v, gbuf, cbuf, ones_v, dstage, deg_acc, sem):
    c = lax.axis_index("c")
    s = lax.axis_index("s")
    np_ = deg_out.shape[1]
    stripe = np_ // NS
    nseg = stripe // CH
    nch = cbuf.shape[0]

    def zv(i, _):
        dstage[pl.ds(i * 16, 16)] = jnp.zeros((16,), jnp.float32)
        return 0
    lax.fori_loop(0, stripe // 16, zv, 0)
    pltpu.sync_copy(dstage, deg_acc.at[pl.ds(s * stripe, stripe)])

    pltpu.sync_copy(nodes_r.at[c, s], idx_v)

    def gnode(j, _):
        pltpu.async_copy(emb.at[idx_v.at[j]], gbuf, sem).wait()
        pltpu.sync_copy(gbuf, feat_out.at[c, pl.ds(s * stripe + j * CH, CH)])
        return 0
    lax.fori_loop(0, nseg, gnode, 0)

    def ov(i, _):
        ones_v[pl.ds(i * 16, 16)] = jnp.ones((16,), jnp.float32)
        return 0
    lax.fori_loop(0, CH // 16, ov, 0)
    pltpu.sync_copy(cols_r.at[c, s], cbuf)
    plsc.subcore_barrier()

    def dscat(j, _):
        pltpu.sync_copy(ones_v, deg_acc.at[cbuf.at[j]], add=True)
        return 0
    lax.fori_loop(0, nch, dscat, 0)
    plsc.subcore_barrier()

    pltpu.sync_copy(deg_acc.at[pl.ds(s * stripe, stripe)], dstage)
    pltpu.sync_copy(dstage, deg_out.at[c, pl.ds(s * stripe, stripe)])


def _agg_body(s_flat, rows_r, cols_r, t_out,
              ridx, cidx, buf, t_acc, sem):
    c = lax.axis_index("c")
    s = lax.axis_index("s")
    np_ = t_out.shape[1]
    stripe = np_ // NS
    nseg = stripe // CH
    nch = ridx.shape[0]

    def zb(i, _):
        r = i // (D // 16)
        k = (i % (D // 16)) * 16
        buf[r, pl.ds(k, 16)] = jnp.zeros((16,), jnp.float32)
        return 0
    lax.fori_loop(0, CH * (D // 16), zb, 0)

    def zt(k, _):
        pltpu.sync_copy(buf, t_acc.at[pl.ds(s * stripe + k * CH, CH)])
        return 0
    lax.fori_loop(0, nseg, zt, 0)

    pltpu.sync_copy(rows_r.at[c, s], ridx)
    pltpu.sync_copy(cols_r.at[c, s], cidx)
    plsc.subcore_barrier()

    def escat(j, _):
        pltpu.async_copy(s_flat.at[ridx.at[j]], buf, sem).wait()
        pltpu.sync_copy(buf, t_acc.at[cidx.at[j]], add=True)
        return 0
    lax.fori_loop(0, nch, escat, 0)
    plsc.subcore_barrier()

    def tout(k, _):
        pltpu.sync_copy(t_acc.at[pl.ds(s * stripe + k * CH, CH)], buf)
        pltpu.sync_copy(buf, t_out.at[c, pl.ds(s * stripe + k * CH, CH)])
        return 0
    lax.fori_loop(0, nseg, tout, 0)


def _tc_a_body(feat_ref, w_ref, dinv_ref, s_ref):
    h = jnp.dot(feat_ref[0], w_ref[...], preferred_element_type=jnp.float32)
    s_ref[0] = h * dinv_ref[0]


def _tc_b_body(t_ref, s_ref, dinv_ref, w_ref, b_ref, x_ref, s2_ref):
    dinv = dinv_ref[0]
    x = jnp.maximum((t_ref[0] + s_ref[0]) * dinv + b_ref[...], 0.0)
    x_ref[0] = x
    h = jnp.dot(x, w_ref[...], preferred_element_type=jnp.float32)
    s2_ref[0] = h * dinv


def _tc_c_body(t_ref, s_ref, dinv_ref, b_ref, x_ref):
    x_ref[0] = jnp.maximum((t_ref[0] + s_ref[0]) * dinv_ref[0] + b_ref[...],
                           0.0)


def kernel(graph_nodes, graph_edges, edge_types, emb, W1, b1, W2, b2):
    del edge_types
    B, N = graph_nodes.shape
    E = graph_edges.shape[2]
    NP = ((N + NS * CH - 1) // (NS * CH)) * (NS * CH)
    ept = E // NS
    eptp = ((ept + 2 * CH - 1) // (2 * CH)) * (2 * CH)
    nch = eptp // CH
    nseg = NP // NS // CH
    mesh = plsc.VectorSubcoreMesh(core_axis_name="c", subcore_axis_name="s")

    nodes_p = jnp.pad(graph_nodes, ((0, 0), (0, NP - N)))
    nodes_r = nodes_p.reshape(B, NS, nseg, CH).astype(jnp.int32)

    rows = graph_edges[:, 0, :].astype(jnp.int32).reshape(B, NS, ept)
    cols = graph_edges[:, 1, :].astype(jnp.int32).reshape(B, NS, ept)
    rows = rows + (jnp.arange(B, dtype=jnp.int32) * NP)[:, None, None]
    rows_r = jnp.pad(rows, ((0, 0), (0, 0), (0, eptp - ept))
                     ).reshape(B, NS, nch, CH)
    padc = (N + (jnp.arange(eptp - ept, dtype=jnp.int32) % (NP - N)))
    padc = jnp.broadcast_to(padc, (B, NS, eptp - ept))
    cols_r = jnp.concatenate([cols, padc], axis=2).reshape(B, NS, nch, CH)

    k1 = pl.kernel(
        _k1_body,
        out_type=[jax.ShapeDtypeStruct((B, NP, D), jnp.float32),
                  jax.ShapeDtypeStruct((B, NP), jnp.float32)],
        mesh=mesh,
        scratch_types=[
            pltpu.VMEM((nseg, CH), jnp.int32),
            pltpu.VMEM((CH, D), jnp.float32),
            pltpu.VMEM((nch, CH), jnp.int32),
            pltpu.VMEM((CH,), jnp.float32),
            pltpu.VMEM((NP // NS,), jnp.float32),
            pltpu.VMEM_SHARED((NP,), jnp.float32),
            pltpu.SemaphoreType.DMA,
        ],
    )
    feat, dege = k1(nodes_r, cols_r, emb)

    dinv3 = lax.rsqrt(jnp.maximum(dege + 1.0, 1.0))[:, :, None]

    agg = pl.kernel(
        _agg_body,
        out_type=jax.ShapeDtypeStruct((B, NP, D), jnp.float32),
        mesh=mesh,
        scratch_types=[
            pltpu.VMEM((nch, CH), jnp.int32),
            pltpu.VMEM((nch, CH), jnp.int32),
            pltpu.VMEM((CH, D), jnp.float32),
            pltpu.VMEM_SHARED((NP, D), jnp.float32),
            pltpu.SemaphoreType.DMA,
        ],
    )

    BN = 1024
    grid = (B, NP // BN)
    blk = pl.BlockSpec((1, BN, D), lambda b, i: (b, i, 0))
    blk1 = pl.BlockSpec((1, BN, 1), lambda b, i: (b, i, 0))
    blkw = pl.BlockSpec((D, D), lambda b, i: (0, 0))
    blkb = pl.BlockSpec((1, D), lambda b, i: (0, 0))

    s1 = pl.pallas_call(
        _tc_a_body,
        grid=grid,
        in_specs=[blk, blkw, blk1],
        out_specs=blk,
        out_shape=jax.ShapeDtypeStruct((B, NP, D), jnp.float32),
    )(feat, W1, dinv3)

    t1 = agg(s1.reshape(B * NP, D), rows_r, cols_r)

    x2, s2 = pl.pallas_call(
        _tc_b_body,
        grid=grid,
        in_specs=[blk, blk, blk1, blkw, blkb],
        out_specs=[blk, blk],
        out_shape=[jax.ShapeDtypeStruct((B, NP, D), jnp.float32),
                   jax.ShapeDtypeStruct((B, NP, D), jnp.float32)],
    )(t1, s1, dinv3, W2, b1.reshape(1, D))

    t2 = agg(s2.reshape(B * NP, D), rows_r, cols_r)

    x3 = pl.pallas_call(
        _tc_c_body,
        grid=grid,
        in_specs=[blk, blk, blk1, blkb],
        out_specs=blk,
        out_shape=jax.ShapeDtypeStruct((B, NP, D), jnp.float32),
    )(t2, s2, dinv3, b2.reshape(1, D))

    x2 = x2[:, :N, :].transpose(0, 2, 1)
    x3 = x3[:, :N, :].transpose(0, 2, 1)
    return jnp.stack([x2, x3], axis=1)


# speedup vs baseline: 1.3924x; 1.3924x over previous
import jax
import jax.numpy as jnp
from jax import lax
from jax.experimental import pallas as pl
from jax.experimental.pallas import tpu as pltpu
from jax.experimental.pallas import tpu_sc as plsc

NS = 16
D = 128
CH = 128


def _k1_body(nodes_r, cols_r, emb, feat_out, deg_out,
             idx_v, gbuf, cbuf, ones_v, dstage, deg_acc, sem):
    c = lax.axis_index("c")
    s = lax.axis_index("s")
    np_ = deg_out.shape[1]
    stripe = np_ // NS
    nseg = stripe // CH
    nch = cbuf.shape[0]

    def zv(i, _):
        dstage[pl.ds(i * 16, 16)] = jnp.zeros((16,), jnp.float32)
        return 0
    lax.fori_loop(0, stripe // 16, zv, 0)
    pltpu.sync_copy(dstage, deg_acc.at[pl.ds(s * stripe, stripe)])

    pltpu.sync_copy(nodes_r.at[c, s], idx_v)

    def gnode(j, _):
        pltpu.async_copy(emb.at[idx_v.at[j]], gbuf, sem).wait()
        pltpu.sync_copy(gbuf, feat_out.at[c, pl.ds(s * stripe + j * CH, CH)])
        return 0
    lax.fori_loop(0, nseg, gnode, 0)

    def ov(i, _):
        ones_v[pl.ds(i * 16, 16)] = jnp.ones((16,), jnp.float32)
        return 0
    lax.fori_loop(0, CH // 16, ov, 0)
    pltpu.sync_copy(cols_r.at[c, s], cbuf)
    plsc.subcore_barrier()

    def dscat(j, _):
        pltpu.sync_copy(ones_v, deg_acc.at[cbuf.at[j]], add=True)
        return 0
    lax.fori_loop(0, nch, dscat, 0)
    plsc.subcore_barrier()

    pltpu.sync_copy(deg_acc.at[pl.ds(s * stripe, stripe)], dstage)
    pltpu.sync_copy(dstage, deg_out.at[c, pl.ds(s * stripe, stripe)])


def _agg_body(s_flat, rows_r, cols_r, t_out,
              ridx, cidx, buf, t_acc, sem):
    c = lax.axis_index("c")
    s = lax.axis_index("s")
    np_ = t_out.shape[1]
    stripe = np_ // NS
    nseg = stripe // CH
    nch = cidx.shape[0]

    def zb(i, _):
        r = i // (D // 16)
        k = (i % (D // 16)) * 16
        buf[r, pl.ds(k, 16)] = jnp.zeros((16,), jnp.float32)
        return 0
    lax.fori_loop(0, CH * (D // 16), zb, 0)

    def zt(k, _):
        pltpu.sync_copy(buf, t_acc.at[pl.ds(s * stripe + k * CH, CH)])
        return 0
    lax.fori_loop(0, nseg, zt, 0)

    pltpu.sync_copy(rows_r.at[c, s], ridx)
    pltpu.sync_copy(cols_r.at[c, s], cidx)
    plsc.subcore_barrier()

    def escat(j, _):
        pltpu.async_copy(s_flat.at[ridx.at[j]], buf, sem).wait()
        pltpu.sync_copy(buf, t_acc.at[cidx.at[j]], add=True)
        return 0
    lax.fori_loop(0, nch, escat, 0)
    plsc.subcore_barrier()

    def tout(k, _):
        pltpu.sync_copy(t_acc.at[pl.ds(s * stripe + k * CH, CH)], buf)
        pltpu.sync_copy(buf, t_out.at[c, pl.ds(s * stripe + k * CH, CH)])
        return 0
    lax.fori_loop(0, nseg, tout, 0)


def _tc_a_body(feat_ref, w_ref, dinv_ref, s_ref):
    h = jnp.dot(feat_ref[0], w_ref[...], preferred_element_type=jnp.float32)
    s_ref[0] = h * dinv_ref[0]


def _tc_b_body(t_ref, s_ref, dinv_ref, w_ref, b_ref, x_ref, s2_ref):
    dinv = dinv_ref[0]
    x = jnp.maximum((t_ref[0] + s_ref[0]) * dinv + b_ref[...], 0.0)
    x_ref[0] = x
    h = jnp.dot(x, w_ref[...], preferred_element_type=jnp.float32)
    s2_ref[0] = h * dinv


def _tc_c_body(t_ref, s_ref, dinv_ref, b_ref, x_ref):
    x_ref[0] = jnp.maximum((t_ref[0] + s_ref[0]) * dinv_ref[0] + b_ref[...],
                           0.0)


def kernel(graph_nodes, graph_edges, edge_types, emb, W1, b1, W2, b2):
    del edge_types
    B, N = graph_nodes.shape
    E = graph_edges.shape[2]
    NP = ((N + NS * CH - 1) // (NS * CH)) * (NS * CH)
    ept = E // NS
    eptp = ((ept + CH - 1) // CH) * CH
    nch = eptp // CH
    nseg = NP // NS // CH
    mesh = plsc.VectorSubcoreMesh(core_axis_name="c", subcore_axis_name="s")

    nodes_p = jnp.pad(graph_nodes, ((0, 0), (0, NP - N)))
    nodes_r = nodes_p.reshape(B, NS, nseg, CH).astype(jnp.int32)

    rows = graph_edges[:, 0, :].astype(jnp.int32).reshape(B, NS, ept)
    cols = graph_edges[:, 1, :].astype(jnp.int32).reshape(B, NS, ept)
    rows = rows + (jnp.arange(B, dtype=jnp.int32) * NP)[:, None, None]
    rows_r = jnp.pad(rows, ((0, 0), (0, 0), (0, eptp - ept))
                     ).reshape(B, NS, nch, CH)
    padc = (N + (jnp.arange(eptp - ept, dtype=jnp.int32) % (NP - N)))
    padc = jnp.broadcast_to(padc, (B, NS, eptp - ept))
    cols_r = jnp.concatenate([cols, padc], axis=2).reshape(B, NS, nch, CH)

    k1 = pl.kernel(
        _k1_body,
        out_type=[jax.ShapeDtypeStruct((B, NP, D), jnp.float32),
                  jax.ShapeDtypeStruct((B, NP), jnp.float32)],
        mesh=mesh,
        scratch_types=[
            pltpu.VMEM((nseg, CH), jnp.int32),
            pltpu.VMEM((CH, D), jnp.float32),
            pltpu.VMEM((nch, CH), jnp.int32),
            pltpu.VMEM((CH,), jnp.float32),
            pltpu.VMEM((NP // NS,), jnp.float32),
            pltpu.VMEM_SHARED((NP,), jnp.float32),
            pltpu.SemaphoreType.DMA,
        ],
    )
    feat, dege = k1(nodes_r, cols_r, emb)

    dinv3 = lax.rsqrt(jnp.maximum(dege + 1.0, 1.0))[:, :, None]

    agg = pl.kernel(
        _agg_body,
        out_type=jax.ShapeDtypeStruct((B, NP, D), jnp.float32),
        mesh=mesh,
        scratch_types=[
            pltpu.VMEM((nch, CH), jnp.int32),
            pltpu.VMEM((nch, CH), jnp.int32),
            pltpu.VMEM((CH, D), jnp.float32),
            pltpu.VMEM_SHARED((NP, D), jnp.float32),
            pltpu.SemaphoreType.DMA,
        ],
    )

    BN = 1024
    grid = (B, NP // BN)
    blk = pl.BlockSpec((1, BN, D), lambda b, i: (b, i, 0))
    blk1 = pl.BlockSpec((1, BN, 1), lambda b, i: (b, i, 0))
    blkw = pl.BlockSpec((D, D), lambda b, i: (0, 0))
    blkb = pl.BlockSpec((1, D), lambda b, i: (0, 0))

    s1 = pl.pallas_call(
        _tc_a_body,
        grid=grid,
        in_specs=[blk, blkw, blk1],
        out_specs=blk,
        out_shape=jax.ShapeDtypeStruct((B, NP, D), jnp.float32),
    )(feat, W1, dinv3)

    t1 = agg(s1.reshape(B * NP, D), rows_r, cols_r)

    x2, s2 = pl.pallas_call(
        _tc_b_body,
        grid=grid,
        in_specs=[blk, blk, blk1, blkw, blkb],
        out_specs=[blk, blk],
        out_shape=[jax.ShapeDtypeStruct((B, NP, D), jnp.float32),
                   jax.ShapeDtypeStruct((B, NP, D), jnp.float32)],
    )(t1, s1, dinv3, W2, b1.reshape(1, D))

    t2 = agg(s2.reshape(B * NP, D), rows_r, cols_r)

    x3 = pl.pallas_call(
        _tc_c_body,
        grid=grid,
        in_specs=[blk, blk, blk1, blkb],
        out_specs=blk,
        out_shape=jax.ShapeDtypeStruct((B, NP, D), jnp.float32),
    )(t2, s2, dinv3, b2.reshape(1, D))

    x2 = x2[:, :N, :].transpose(0, 2, 1)
    x3 = x3[:, :N, :].transpose(0, 2, 1)
    return jnp.stack([x2, x3], axis=1)
